# baseline (speedup 1.0000x reference)
import functools

import jax
import jax.numpy as jnp
from jax.experimental import pallas as pl
from jax.experimental.pallas import tpu as pltpu

_VMEM_LIMIT = pltpu.CompilerParams(vmem_limit_bytes=63 * 1024 * 1024)


def _pick_blk(n, cap):
    for blk in (cap, 400, 80, 16):
        if blk <= cap and n % blk == 0:
            return blk
    return n


def _prep_kernel(adj_ref, mask_ref, dinv_ref, *, nblk, blk):
    j = pl.program_id(0)
    a = adj_ref[...]
    mask_ref[...] = a.astype(jnp.bfloat16)

    @pl.when(j == 0)
    def _():
        dinv_ref[...] = jnp.ones_like(dinv_ref)

    ones = jnp.ones((blk, 1), dtype=jnp.float32)
    dinv_ref[...] += jax.lax.dot_general(
        a, ones, (((0,), (0,)), ((), ())),
        preferred_element_type=jnp.float32)

    @pl.when(j == nblk - 1)
    def _():
        d = dinv_ref[...]
        dinv_ref[...] = jnp.where(d > 0, jax.lax.rsqrt(d), 0.0)


def _gcn_kernel(mask_ref, x_ref, w_ref, b_ref, dinv_ref, out_ref, u_ref,
                *, nblk, blk, relu, logsm, temp):
    j = pl.program_id(0)

    @pl.when(j == 0)
    def _():
        u = dinv_ref[...] * jnp.dot(
            x_ref[...], w_ref[...], preferred_element_type=jnp.float32)
        u_ref[...] = u.astype(jnp.bfloat16)
        out_ref[...] = jnp.zeros_like(out_ref)

    u_blk = u_ref[pl.ds(j * blk, blk), :]
    out_ref[...] += jax.lax.dot_general(
        mask_ref[...], u_blk, (((0,), (0,)), ((), ())),
        preferred_element_type=jnp.float32)

    @pl.when(j == nblk - 1)
    def _():
        v = dinv_ref[...] * (out_ref[...] + u_ref[...].astype(jnp.float32))
        v = v + b_ref[...]
        if relu:
            v = jnp.maximum(v, 0.0)
        if logsm:
            t = v * (1.0 / temp)
            m = jnp.max(t, axis=1, keepdims=True)
            s = t - m
            v = s - jnp.log(jnp.sum(jnp.exp(s), axis=1, keepdims=True))
        out_ref[...] = v


def kernel(x, adj, W1, b1, W2, b2):
    n = adj.shape[0]
    pblk = _pick_blk(n, 80)
    pnblk = n // pblk

    mask, dinv = pl.pallas_call(
        functools.partial(_prep_kernel, nblk=pnblk, blk=pblk),
        grid=(pnblk,),
        in_specs=[pl.BlockSpec((pblk, n), lambda j: (j, 0))],
        out_specs=[
            pl.BlockSpec((pblk, n), lambda j: (j, 0)),
            pl.BlockSpec((n, 1), lambda j: (0, 0)),
        ],
        out_shape=[
            jax.ShapeDtypeStruct((n, n), jnp.bfloat16),
            jax.ShapeDtypeStruct((n, 1), jnp.float32),
        ],
        compiler_params=_VMEM_LIMIT,
    )(adj)

    blk = _pick_blk(n, 400)
    nblk = n // blk

    def layer(h, w, b, relu, logsm, temp):
        f = w.shape[1]
        return pl.pallas_call(
            functools.partial(_gcn_kernel, nblk=nblk, blk=blk, relu=relu,
                              logsm=logsm, temp=temp),
            grid=(nblk,),
            in_specs=[
                pl.BlockSpec((blk, n), lambda j: (j, 0)),
                pl.BlockSpec((n, h.shape[1]), lambda j: (0, 0)),
                pl.BlockSpec(w.shape, lambda j: (0, 0)),
                pl.BlockSpec((1, f), lambda j: (0, 0)),
                pl.BlockSpec((n, 1), lambda j: (0, 0)),
            ],
            out_specs=pl.BlockSpec((n, f), lambda j: (0, 0)),
            out_shape=jax.ShapeDtypeStruct((n, f), jnp.float32),
            scratch_shapes=[pltpu.VMEM((n, f), jnp.bfloat16)],
            compiler_params=_VMEM_LIMIT,
        )(mask, h, w, b.reshape(1, f), dinv)

    h1 = layer(x, W1, b1, relu=True, logsm=False, temp=1.0)
    out = layer(h1, W2, b2, relu=False, logsm=True, temp=0.2)
    return out


# speedup vs baseline: 1.2601x; 1.2601x over previous
import functools

import jax
import jax.numpy as jnp
from jax.experimental import pallas as pl
from jax.experimental.pallas import tpu as pltpu

_VMEM_LIMIT = pltpu.CompilerParams(vmem_limit_bytes=63 * 1024 * 1024)


def _pick_blk(n):
    for blk in (400, 80, 16):
        if n % blk == 0:
            return blk
    return n


def _prep_kernel(adj_ref, mask_ref, dinv_ref, *, nblk, blk):
    j = pl.program_id(0)
    a = adj_ref[...]
    mask_ref[...] = a.astype(jnp.bfloat16)

    @pl.when(j == 0)
    def _():
        dinv_ref[...] = jnp.ones_like(dinv_ref)

    ones = jnp.ones((1, blk), dtype=jnp.float32)
    dinv_ref[...] += jax.lax.dot_general(
        ones, a, (((1,), (0,)), ((), ())),
        preferred_element_type=jnp.float32)

    @pl.when(j == nblk - 1)
    def _():
        d = dinv_ref[...]
        dinv_ref[...] = jnp.where(d > 0, jax.lax.rsqrt(d), 0.0)


def _gcn_kernel(mask_ref, x_ref, w_ref, b_ref, dinv_ref, out_ref,
                u_ref, dcol_ref, *, nblk, blk, relu, logsm, temp):
    j = pl.program_id(0)

    @pl.when(j == 0)
    def _():
        one = jnp.ones((1, 1), dtype=jnp.float32)
        dcol_ref[...] = jax.lax.dot_general(
            dinv_ref[...], one, (((0,), (0,)), ((), ())),
            preferred_element_type=jnp.float32)
        u = dcol_ref[...] * jnp.dot(
            x_ref[...], w_ref[...], preferred_element_type=jnp.float32)
        u_ref[...] = u.astype(jnp.bfloat16)
        out_ref[...] = jnp.zeros_like(out_ref)

    u_blk = u_ref[pl.ds(j * blk, blk), :]
    out_ref[...] += jax.lax.dot_general(
        mask_ref[...], u_blk, (((0,), (0,)), ((), ())),
        preferred_element_type=jnp.float32)

    @pl.when(j == nblk - 1)
    def _():
        v = dcol_ref[...] * (out_ref[...] + u_ref[...].astype(jnp.float32))
        v = v + b_ref[...]
        if relu:
            v = jnp.maximum(v, 0.0)
        if logsm:
            t = v * (1.0 / temp)
            m = jnp.max(t, axis=1, keepdims=True)
            s = t - m
            v = s - jnp.log(jnp.sum(jnp.exp(s), axis=1, keepdims=True))
        out_ref[...] = v


def kernel(x, adj, W1, b1, W2, b2):
    n = adj.shape[0]
    blk = _pick_blk(n)
    nblk = n // blk

    mask, dinv = pl.pallas_call(
        functools.partial(_prep_kernel, nblk=nblk, blk=blk),
        grid=(nblk,),
        in_specs=[pl.BlockSpec((blk, n), lambda j: (j, 0))],
        out_specs=[
            pl.BlockSpec((blk, n), lambda j: (j, 0)),
            pl.BlockSpec((1, n), lambda j: (0, 0)),
        ],
        out_shape=[
            jax.ShapeDtypeStruct((n, n), jnp.bfloat16),
            jax.ShapeDtypeStruct((1, n), jnp.float32),
        ],
        compiler_params=_VMEM_LIMIT,
    )(adj)

    def layer(h, w, b, relu, logsm, temp):
        f = w.shape[1]
        return pl.pallas_call(
            functools.partial(_gcn_kernel, nblk=nblk, blk=blk, relu=relu,
                              logsm=logsm, temp=temp),
            grid=(nblk,),
            in_specs=[
                pl.BlockSpec((blk, n), lambda j: (j, 0)),
                pl.BlockSpec((n, h.shape[1]), lambda j: (0, 0)),
                pl.BlockSpec(w.shape, lambda j: (0, 0)),
                pl.BlockSpec((1, f), lambda j: (0, 0)),
                pl.BlockSpec((1, n), lambda j: (0, 0)),
            ],
            out_specs=pl.BlockSpec((n, f), lambda j: (0, 0)),
            out_shape=jax.ShapeDtypeStruct((n, f), jnp.float32),
            scratch_shapes=[
                pltpu.VMEM((n, f), jnp.bfloat16),
                pltpu.VMEM((n, 1), jnp.float32),
            ],
            compiler_params=_VMEM_LIMIT,
        )(mask, h, w, b.reshape(1, f), dinv)

    h1 = layer(x, W1, b1, relu=True, logsm=False, temp=1.0)
    out = layer(h1, W2, b2, relu=False, logsm=True, temp=0.2)
    return out


# speedup vs baseline: 1.4187x; 1.1258x over previous
import functools

import jax
import jax.numpy as jnp
from jax.experimental import pallas as pl
from jax.experimental.pallas import tpu as pltpu

_VMEM_LIMIT = pltpu.CompilerParams(vmem_limit_bytes=63 * 1024 * 1024)


def _pick_blk(n):
    for blk in (400, 80, 16):
        if n % blk == 0:
            return blk
    return n


def _prep_kernel(adj_ref, mask_ref, dinv_ref, *, nblk, blk):
    j = pl.program_id(0)
    a = adj_ref[...]
    mask_ref[...] = a.astype(jnp.int8)

    @pl.when(j == 0)
    def _():
        dinv_ref[...] = jnp.ones_like(dinv_ref)

    ones = jnp.ones((1, blk), dtype=jnp.float32)
    dinv_ref[...] += jax.lax.dot_general(
        ones, a, (((1,), (0,)), ((), ())),
        preferred_element_type=jnp.float32)

    @pl.when(j == nblk - 1)
    def _():
        d = dinv_ref[...]
        dinv_ref[...] = jnp.where(d > 0, jax.lax.rsqrt(d), 0.0)


def _gcn_kernel(mask_ref, x_ref, w_ref, b_ref, dinv_ref, out_ref,
                u_ref, dcol_ref, *, nblk, blk, relu, logsm, temp):
    j = pl.program_id(0)

    @pl.when(j == 0)
    def _():
        one = jnp.ones((1, 1), dtype=jnp.float32)
        dcol_ref[...] = jax.lax.dot_general(
            dinv_ref[...], one, (((0,), (0,)), ((), ())),
            preferred_element_type=jnp.float32)
        u = dcol_ref[...] * jnp.dot(
            x_ref[...], w_ref[...], preferred_element_type=jnp.float32)
        u_ref[...] = u.astype(jnp.bfloat16)
        out_ref[...] = jnp.zeros_like(out_ref)

    u_blk = u_ref[pl.ds(j * blk, blk), :]
    out_ref[...] += jax.lax.dot_general(
        mask_ref[...].astype(jnp.bfloat16), u_blk, (((0,), (0,)), ((), ())),
        preferred_element_type=jnp.float32)

    @pl.when(j == nblk - 1)
    def _():
        v = dcol_ref[...] * (out_ref[...] + u_ref[...].astype(jnp.float32))
        v = v + b_ref[...]
        if relu:
            v = jnp.maximum(v, 0.0)
        if logsm:
            t = v * (1.0 / temp)
            m = jnp.max(t, axis=1, keepdims=True)
            s = t - m
            v = s - jnp.log(jnp.sum(jnp.exp(s), axis=1, keepdims=True))
        out_ref[...] = v


def kernel(x, adj, W1, b1, W2, b2):
    n = adj.shape[0]
    blk = _pick_blk(n)
    nblk = n // blk

    mask, dinv = pl.pallas_call(
        functools.partial(_prep_kernel, nblk=nblk, blk=blk),
        grid=(nblk,),
        in_specs=[pl.BlockSpec((blk, n), lambda j: (j, 0))],
        out_specs=[
            pl.BlockSpec((blk, n), lambda j: (j, 0)),
            pl.BlockSpec((1, n), lambda j: (0, 0)),
        ],
        out_shape=[
            jax.ShapeDtypeStruct((n, n), jnp.int8),
            jax.ShapeDtypeStruct((1, n), jnp.float32),
        ],
        compiler_params=_VMEM_LIMIT,
    )(adj)

    def layer(h, w, b, relu, logsm, temp):
        f = w.shape[1]
        return pl.pallas_call(
            functools.partial(_gcn_kernel, nblk=nblk, blk=blk, relu=relu,
                              logsm=logsm, temp=temp),
            grid=(nblk,),
            in_specs=[
                pl.BlockSpec((blk, n), lambda j: (j, 0)),
                pl.BlockSpec((n, h.shape[1]), lambda j: (0, 0)),
                pl.BlockSpec(w.shape, lambda j: (0, 0)),
                pl.BlockSpec((1, f), lambda j: (0, 0)),
                pl.BlockSpec((1, n), lambda j: (0, 0)),
            ],
            out_specs=pl.BlockSpec((n, f), lambda j: (0, 0)),
            out_shape=jax.ShapeDtypeStruct((n, f), jnp.float32),
            scratch_shapes=[
                pltpu.VMEM((n, f), jnp.bfloat16),
                pltpu.VMEM((n, 1), jnp.float32),
            ],
            compiler_params=_VMEM_LIMIT,
        )(mask, h, w, b.reshape(1, f), dinv)

    h1 = layer(x, W1, b1, relu=True, logsm=False, temp=1.0)
    out = layer(h1, W2, b2, relu=False, logsm=True, temp=0.2)
    return out


# speedup vs baseline: 1.5609x; 1.1003x over previous
import functools

import jax
import jax.numpy as jnp
from jax.experimental import pallas as pl
from jax.experimental.pallas import tpu as pltpu

_VMEM_LIMIT = pltpu.CompilerParams(vmem_limit_bytes=63 * 1024 * 1024)


def _pick_blk(n):
    for blk in (400, 80, 16):
        if n % blk == 0:
            return blk
    return n


def _prep_kernel(adj_ref, mask_ref, dinv_ref, *, nblk, blk):
    j = pl.program_id(0)
    a = adj_ref[...]
    mask_ref[...] = a.astype(jnp.int8)

    @pl.when(j == 0)
    def _():
        dinv_ref[...] = jnp.ones_like(dinv_ref)

    ones = jnp.ones((1, blk), dtype=jnp.float32)
    dinv_ref[...] += jax.lax.dot_general(
        ones, a, (((1,), (0,)), ((), ())),
        preferred_element_type=jnp.float32)

    @pl.when(j == nblk - 1)
    def _():
        d = dinv_ref[...]
        dinv_ref[...] = jnp.where(d > 0, jax.lax.rsqrt(d), 0.0)


def _gcn_kernel(mask_ref, x_ref, w_ref, b_ref, dinv_ref, out_ref,
                u_ref, dcol_ref, *, nblk, blk, relu, logsm, temp):
    j = pl.program_id(0)

    @pl.when(j == 0)
    def _():
        one = jnp.ones((1, 1), dtype=jnp.float32)
        dcol_ref[...] = jax.lax.dot_general(
            dinv_ref[...], one, (((0,), (0,)), ((), ())),
            preferred_element_type=jnp.float32)
        u = dcol_ref[...] * jnp.dot(
            x_ref[...], w_ref[...], preferred_element_type=jnp.float32)
        u_ref[...] = u.astype(jnp.bfloat16)
        out_ref[...] = jnp.zeros_like(out_ref)

    u_blk = u_ref[pl.ds(j * blk, blk), :]
    out_ref[...] += jax.lax.dot_general(
        mask_ref[...].astype(jnp.bfloat16), u_blk, (((0,), (0,)), ((), ())),
        preferred_element_type=jnp.float32)

    @pl.when(j == nblk - 1)
    def _():
        v = dcol_ref[...] * (out_ref[...] + u_ref[...].astype(jnp.float32))
        v = v + b_ref[...]
        if relu:
            v = jnp.maximum(v, 0.0)
        if logsm:
            t = v * (1.0 / temp)
            m = jnp.max(t, axis=1, keepdims=True)
            s = t - m
            v = s - jnp.log(jnp.sum(jnp.exp(s), axis=1, keepdims=True))
        out_ref[...] = v


def kernel(x, adj, W1, b1, W2, b2):
    n = adj.shape[0]
    blk = _pick_blk(n)
    nblk = n // blk

    mask, dinv = pl.pallas_call(
        functools.partial(_prep_kernel, nblk=nblk, blk=blk),
        grid=(nblk,),
        in_specs=[pl.BlockSpec((blk, n), lambda j: (j, 0))],
        out_specs=[
            pl.BlockSpec((blk, n), lambda j: (j, 0)),
            pl.BlockSpec((1, n), lambda j: (0, 0)),
        ],
        out_shape=[
            jax.ShapeDtypeStruct((n, n), jnp.int8),
            jax.ShapeDtypeStruct((1, n), jnp.float32),
        ],
        compiler_params=_VMEM_LIMIT,
    )(adj)

    lblk = 1000 if n % 1000 == 0 else blk
    lnblk = n // lblk

    def layer(h, w, b, relu, logsm, temp):
        f = w.shape[1]
        return pl.pallas_call(
            functools.partial(_gcn_kernel, nblk=lnblk, blk=lblk, relu=relu,
                              logsm=logsm, temp=temp),
            grid=(lnblk,),
            in_specs=[
                pl.BlockSpec((lblk, n), lambda j: (j, 0)),
                pl.BlockSpec((n, h.shape[1]), lambda j: (0, 0)),
                pl.BlockSpec(w.shape, lambda j: (0, 0)),
                pl.BlockSpec((1, f), lambda j: (0, 0)),
                pl.BlockSpec((1, n), lambda j: (0, 0)),
            ],
            out_specs=pl.BlockSpec((n, f), lambda j: (0, 0)),
            out_shape=jax.ShapeDtypeStruct((n, f), jnp.float32),
            scratch_shapes=[
                pltpu.VMEM((n, f), jnp.bfloat16),
                pltpu.VMEM((n, 1), jnp.float32),
            ],
            compiler_params=_VMEM_LIMIT,
        )(mask, h, w, b.reshape(1, f), dinv)

    h1 = layer(x, W1, b1, relu=True, logsm=False, temp=1.0)
    out = layer(h1, W2, b2, relu=False, logsm=True, temp=0.2)
    return out
